# baseline (speedup 1.0000x reference)
import functools

import jax
import jax.numpy as jnp
from jax import lax
from jax.experimental import pallas as pl
from jax.experimental.pallas import tpu as pltpu
from jax.experimental.pallas import tpu_sc as plsc

DIM = 4096
E = 64
K = 8
N = 16384
BLK = 1024
GRID = N // BLK
FLAT = N * K
NW = 32
SUB = 16
CHUNK = FLAT // SUB
OUTW = FLAT // NW


def _router_block(x_ref, w_ref, topn_ref, packed_ref, counts_ref,
                  carry_ref):
    b = pl.program_id(0)

    @pl.when(b == 0)
    def _():
        carry_ref[...] = jnp.zeros_like(carry_ref)

    logits = jax.lax.dot_general(
        x_ref[...], w_ref[...], (((1,), (1,)), ((), ())),
        preferred_element_type=jnp.float32)
    m = jnp.max(logits, axis=1, keepdims=True)
    p = jnp.exp(logits - m)
    probs = p / jnp.sum(p, axis=1, keepdims=True)

    lane = lax.broadcasted_iota(jnp.int32, (BLK, E), 1)
    cur = probs
    vals, sels, ohs = [], [], []
    for _ in range(K):
        mx = jnp.max(cur, axis=1, keepdims=True)
        am = jnp.argmax(cur, axis=1, keepdims=True)
        oh = lane == am
        vals.append(mx)
        sels.append(am)
        ohs.append(oh)
        cur = jnp.where(oh, -1.0, cur)

    top = jnp.concatenate(vals, axis=1)
    denom = jnp.clip(jnp.sum(top, axis=1, keepdims=True), 1e-8, None)
    topn_ref[...] = top / denom
    sel = jnp.concatenate(sels, axis=1)

    hot = jnp.zeros((BLK, E), jnp.float32)
    for oh in ohs:
        hot = hot + oh.astype(jnp.float32)
    ri = lax.broadcasted_iota(jnp.int32, (BLK, BLK), 0)
    ci = lax.broadcasted_iota(jnp.int32, (BLK, BLK), 1)
    ltri = (ri > ci).astype(jnp.float32)
    within = jnp.dot(ltri, hot, preferred_element_type=jnp.float32)
    ctot = within + carry_ref[...]
    occ_cols = [jnp.sum(jnp.where(oh, ctot, 0.0), axis=1, keepdims=True)
                for oh in ohs]
    occ = jnp.concatenate(occ_cols, axis=1).astype(jnp.int32)
    packed_ref[...] = (occ << 6) | sel
    carry_ref[...] = carry_ref[...] + jnp.sum(hot, axis=0, keepdims=True)
    counts_ref[...] = carry_ref[...].astype(jnp.int32)


def _router(x, wt):
    return pl.pallas_call(
        _router_block,
        grid=(GRID,),
        in_specs=[
            pl.BlockSpec((BLK, DIM), lambda b: (b, 0)),
            pl.BlockSpec((E, DIM), lambda b: (0, 0)),
        ],
        out_specs=[
            pl.BlockSpec((BLK, K), lambda b: (b, 0)),
            pl.BlockSpec((BLK, K), lambda b: (b, 0)),
            pl.BlockSpec((1, E), lambda b: (0, 0)),
        ],
        out_shape=[
            jax.ShapeDtypeStruct((N, K), jnp.float32),
            jax.ShapeDtypeStruct((N, K), jnp.int32),
            jax.ShapeDtypeStruct((1, E), jnp.int32),
        ],
        scratch_shapes=[pltpu.VMEM((1, E), jnp.float32)],
    )(x, wt)


def _sc_body(pk_hbm, sc_hbm, counts_hbm, out_sc, out_tok,
             pk_v, sc_v, rank_v, tok_v, counts_v, prefix_v,
             sh_sc, sh_tok, sem):
    cid = lax.axis_index("c")
    sid = lax.axis_index("s")
    base = sid * CHUNK
    pltpu.sync_copy(counts_hbm, counts_v)
    pltpu.sync_copy(pk_hbm.at[pl.ds(base, CHUNK)], pk_v)
    pltpu.sync_copy(sc_hbm.at[pl.ds(base, CHUNK)], sc_v)

    carry = jnp.int32(0)
    for i in range(E // 16):
        v = counts_v[pl.ds(16 * i, 16)]
        inc = plsc.cumsum(v)
        prefix_v[pl.ds(16 * i, 16)] = (inc - v) + carry
        carry = carry + jnp.sum(v)

    def step(i, c):
        off = i * 16
        pk = pk_v[pl.ds(off, 16)]
        e = pk & 63
        occ = lax.shift_right_logical(pk, 6)
        pre = plsc.load_gather(prefix_v, [e])
        rank_v[pl.ds(off, 16)] = pre + occ
        flat = base + off + lax.iota(jnp.int32, 16)
        tok_v[pl.ds(off, 16)] = lax.shift_right_logical(flat, 3)
        return c

    lax.fori_loop(0, CHUNK // 16, step, 0, unroll=4)
    pltpu.sync_copy(sc_v, sh_sc.at[rank_v])
    pltpu.sync_copy(tok_v, sh_tok.at[rank_v])
    plsc.subcore_barrier()
    ob = cid * (FLAT // 2) + sid * OUTW
    pltpu.sync_copy(sh_sc.at[pl.ds(ob, OUTW)], out_sc.at[pl.ds(ob, OUTW)])
    pltpu.sync_copy(sh_tok.at[pl.ds(ob, OUTW)], out_tok.at[pl.ds(ob, OUTW)])


@functools.cache
def _sc_scatter():
    return functools.partial(
        pl.kernel,
        mesh=plsc.VectorSubcoreMesh(core_axis_name="c", subcore_axis_name="s"),
        out_type=[
            jax.ShapeDtypeStruct((FLAT,), jnp.float32),
            jax.ShapeDtypeStruct((FLAT,), jnp.int32),
        ],
        scratch_types=[
            pltpu.VMEM((CHUNK,), jnp.int32),
            pltpu.VMEM((CHUNK,), jnp.float32),
            pltpu.VMEM((CHUNK,), jnp.int32),
            pltpu.VMEM((CHUNK,), jnp.int32),
            pltpu.VMEM((E,), jnp.int32),
            pltpu.VMEM((E,), jnp.int32),
            pltpu.VMEM_SHARED((FLAT,), jnp.float32),
            pltpu.VMEM_SHARED((FLAT,), jnp.int32),
            pltpu.SemaphoreType.DMA,
        ],
        compiler_params=pltpu.CompilerParams(needs_layout_passes=False),
    )(_sc_body)


def kernel(x, W_gate):
    topn, packed, counts2 = _router(x, W_gate)
    counts = counts2.reshape(E)
    out_sc, out_tok = _sc_scatter()(
        packed.reshape(FLAT), topn.reshape(FLAT), counts)
    return out_sc, out_tok, counts


# speedup vs baseline: 1.0001x; 1.0001x over previous
import functools

import jax
import jax.numpy as jnp
from jax import lax
from jax.experimental import pallas as pl
from jax.experimental.pallas import tpu as pltpu
from jax.experimental.pallas import tpu_sc as plsc

DIM = 4096
E = 64
K = 8
N = 16384
BLK = 512
GRID = N // BLK
FLAT = N * K
NW = 32
SUB = 16
CHUNK = FLAT // SUB
OUTW = FLAT // NW


def _router_block(x_ref, w_ref, topn_ref, packed_ref, counts_ref,
                  carry_ref):
    b = pl.program_id(0)

    @pl.when(b == 0)
    def _():
        carry_ref[...] = jnp.zeros_like(carry_ref)

    logits = jax.lax.dot_general(
        x_ref[...], w_ref[...], (((1,), (1,)), ((), ())),
        preferred_element_type=jnp.float32)
    m = jnp.max(logits, axis=1, keepdims=True)
    p = jnp.exp(logits - m)
    probs = p / jnp.sum(p, axis=1, keepdims=True)

    lane = lax.broadcasted_iota(jnp.int32, (BLK, E), 1)
    cur = probs
    vals, sels, ohs = [], [], []
    for _ in range(K):
        mx = jnp.max(cur, axis=1, keepdims=True)
        am = jnp.argmax(cur, axis=1, keepdims=True)
        oh = lane == am
        vals.append(mx)
        sels.append(am)
        ohs.append(oh)
        cur = jnp.where(oh, -1.0, cur)

    top = jnp.concatenate(vals, axis=1)
    denom = jnp.clip(jnp.sum(top, axis=1, keepdims=True), 1e-8, None)
    topn_ref[...] = top / denom
    sel = jnp.concatenate(sels, axis=1)

    hot = jnp.zeros((BLK, E), jnp.float32)
    for oh in ohs:
        hot = hot + oh.astype(jnp.float32)
    ri = lax.broadcasted_iota(jnp.int32, (BLK, BLK), 0)
    ci = lax.broadcasted_iota(jnp.int32, (BLK, BLK), 1)
    ltri = (ri > ci).astype(jnp.float32)
    within = jnp.dot(ltri, hot, preferred_element_type=jnp.float32)
    ctot = within + carry_ref[...]
    occ_cols = [jnp.sum(jnp.where(oh, ctot, 0.0), axis=1, keepdims=True)
                for oh in ohs]
    occ = jnp.concatenate(occ_cols, axis=1).astype(jnp.int32)
    packed_ref[...] = (occ << 6) | sel
    carry_ref[...] = carry_ref[...] + jnp.sum(hot, axis=0, keepdims=True)
    counts_ref[...] = carry_ref[...].astype(jnp.int32)


def _router(x, wt):
    return pl.pallas_call(
        _router_block,
        grid=(GRID,),
        in_specs=[
            pl.BlockSpec((BLK, DIM), lambda b: (b, 0)),
            pl.BlockSpec((E, DIM), lambda b: (0, 0)),
        ],
        out_specs=[
            pl.BlockSpec((BLK, K), lambda b: (b, 0)),
            pl.BlockSpec((BLK, K), lambda b: (b, 0)),
            pl.BlockSpec((1, E), lambda b: (0, 0)),
        ],
        out_shape=[
            jax.ShapeDtypeStruct((N, K), jnp.float32),
            jax.ShapeDtypeStruct((N, K), jnp.int32),
            jax.ShapeDtypeStruct((1, E), jnp.int32),
        ],
        scratch_shapes=[pltpu.VMEM((1, E), jnp.float32)],
    )(x, wt)


def _sc_body(pk_hbm, sc_hbm, counts_hbm, out_sc, out_tok,
             pk_v, sc_v, rank_v, tok_v, counts_v, prefix_v,
             sh_sc, sh_tok, sem):
    cid = lax.axis_index("c")
    sid = lax.axis_index("s")
    base = sid * CHUNK
    pltpu.sync_copy(counts_hbm, counts_v)
    pltpu.sync_copy(pk_hbm.at[pl.ds(base, CHUNK)], pk_v)
    pltpu.sync_copy(sc_hbm.at[pl.ds(base, CHUNK)], sc_v)

    carry = jnp.int32(0)
    for i in range(E // 16):
        v = counts_v[pl.ds(16 * i, 16)]
        inc = plsc.cumsum(v)
        prefix_v[pl.ds(16 * i, 16)] = (inc - v) + carry
        carry = carry + jnp.sum(v)

    def step(i, c):
        off = i * 16
        pk = pk_v[pl.ds(off, 16)]
        e = pk & 63
        occ = lax.shift_right_logical(pk, 6)
        pre = plsc.load_gather(prefix_v, [e])
        rank_v[pl.ds(off, 16)] = pre + occ
        flat = base + off + lax.iota(jnp.int32, 16)
        tok_v[pl.ds(off, 16)] = lax.shift_right_logical(flat, 3)
        return c

    lax.fori_loop(0, CHUNK // 16, step, 0, unroll=4)
    pltpu.sync_copy(sc_v, sh_sc.at[rank_v])
    pltpu.sync_copy(tok_v, sh_tok.at[rank_v])
    plsc.subcore_barrier()
    ob = cid * (FLAT // 2) + sid * OUTW
    pltpu.sync_copy(sh_sc.at[pl.ds(ob, OUTW)], out_sc.at[pl.ds(ob, OUTW)])
    pltpu.sync_copy(sh_tok.at[pl.ds(ob, OUTW)], out_tok.at[pl.ds(ob, OUTW)])


@functools.cache
def _sc_scatter():
    return functools.partial(
        pl.kernel,
        mesh=plsc.VectorSubcoreMesh(core_axis_name="c", subcore_axis_name="s"),
        out_type=[
            jax.ShapeDtypeStruct((FLAT,), jnp.float32),
            jax.ShapeDtypeStruct((FLAT,), jnp.int32),
        ],
        scratch_types=[
            pltpu.VMEM((CHUNK,), jnp.int32),
            pltpu.VMEM((CHUNK,), jnp.float32),
            pltpu.VMEM((CHUNK,), jnp.int32),
            pltpu.VMEM((CHUNK,), jnp.int32),
            pltpu.VMEM((E,), jnp.int32),
            pltpu.VMEM((E,), jnp.int32),
            pltpu.VMEM_SHARED((FLAT,), jnp.float32),
            pltpu.VMEM_SHARED((FLAT,), jnp.int32),
            pltpu.SemaphoreType.DMA,
        ],
        compiler_params=pltpu.CompilerParams(needs_layout_passes=False),
    )(_sc_body)


def kernel(x, W_gate):
    topn, packed, counts2 = _router(x, W_gate)
    counts = counts2.reshape(E)
    out_sc, out_tok = _sc_scatter()(
        packed.reshape(FLAT), topn.reshape(FLAT), counts)
    return out_sc, out_tok, counts


# speedup vs baseline: 1.0076x; 1.0075x over previous
import functools

import jax
import jax.numpy as jnp
from jax import lax
from jax.experimental import pallas as pl
from jax.experimental.pallas import tpu as pltpu
from jax.experimental.pallas import tpu_sc as plsc

DIM = 4096
E = 64
K = 8
N = 16384
BLK = 512
GRID = N // BLK
FLAT = N * K
NW = 32
SUB = 16
CHUNK = FLAT // SUB
OUTW = FLAT // NW


def _router_block(x_ref, w_ref, topn_ref, packed_ref, counts_ref,
                  carry_ref):
    b = pl.program_id(0)

    @pl.when(b == 0)
    def _():
        carry_ref[...] = jnp.zeros_like(carry_ref)

    logits = jax.lax.dot_general(
        x_ref[...], w_ref[...], (((1,), (1,)), ((), ())),
        preferred_element_type=jnp.float32)
    m = jnp.max(logits, axis=1, keepdims=True)
    p = jnp.exp(logits - m)
    probs = p / jnp.sum(p, axis=1, keepdims=True)

    lane = lax.broadcasted_iota(jnp.int32, (BLK, E), 1)
    cur = probs
    vals, sels, ohs = [], [], []
    for _ in range(K):
        mx = jnp.max(cur, axis=1, keepdims=True)
        am = jnp.argmax(cur, axis=1, keepdims=True)
        oh = lane == am
        vals.append(mx)
        sels.append(am)
        ohs.append(oh)
        cur = jnp.where(oh, -1.0, cur)

    top = jnp.concatenate(vals, axis=1)
    denom = jnp.clip(jnp.sum(top, axis=1, keepdims=True), 1e-8, None)
    topn_ref[...] = top / denom
    sel = jnp.concatenate(sels, axis=1)

    hot = jnp.zeros((BLK, E), jnp.float32)
    for oh in ohs:
        hot = hot + oh.astype(jnp.float32)
    ri = lax.broadcasted_iota(jnp.int32, (BLK, BLK), 0)
    ci = lax.broadcasted_iota(jnp.int32, (BLK, BLK), 1)
    ltri = (ri > ci).astype(jnp.float32)
    within = jnp.dot(ltri, hot, preferred_element_type=jnp.float32)
    ctot = within + carry_ref[...]
    occ_cols = [jnp.sum(jnp.where(oh, ctot, 0.0), axis=1, keepdims=True)
                for oh in ohs]
    occ = jnp.concatenate(occ_cols, axis=1).astype(jnp.int32)
    packed_ref[...] = (occ << 6) | sel
    carry_ref[...] = carry_ref[...] + jnp.sum(hot, axis=0, keepdims=True)
    counts_ref[...] = carry_ref[...].astype(jnp.int32)


def _router(x, wt):
    return pl.pallas_call(
        _router_block,
        grid=(GRID,),
        in_specs=[
            pl.BlockSpec((BLK, DIM), lambda b: (b, 0)),
            pl.BlockSpec((E, DIM), lambda b: (0, 0)),
        ],
        out_specs=[
            pl.BlockSpec((BLK, K), lambda b: (b, 0)),
            pl.BlockSpec((BLK, K), lambda b: (b, 0)),
            pl.BlockSpec((1, E), lambda b: (0, 0)),
        ],
        out_shape=[
            jax.ShapeDtypeStruct((N, K), jnp.float32),
            jax.ShapeDtypeStruct((N, K), jnp.int32),
            jax.ShapeDtypeStruct((1, E), jnp.int32),
        ],
        scratch_shapes=[pltpu.VMEM((1, E), jnp.float32)],
    )(x, wt)


def _sc_body(pk_hbm, sc_hbm, counts_hbm, out_sc, out_tok, counts_out,
             pk_v, sc_v, rank_v, tok_v, counts_v, prefix_v,
             sh_sc, sh_tok, sem_a, sem_b):
    cid = lax.axis_index("c")
    sid = lax.axis_index("s")
    base = sid * CHUNK
    cpk = pltpu.make_async_copy(pk_hbm.at[pl.ds(base, CHUNK)], pk_v, sem_a)
    cpk.start()
    csc = pltpu.make_async_copy(sc_hbm.at[pl.ds(base, CHUNK)], sc_v, sem_b)
    csc.start()
    pltpu.sync_copy(counts_hbm, counts_v)

    @pl.when(jnp.logical_and(cid == 0, sid == 0))
    def _():
        pltpu.sync_copy(counts_v, counts_out)

    carry = jnp.int32(0)
    for i in range(E // 16):
        v = counts_v[pl.ds(16 * i, 16)]
        inc = plsc.cumsum(v)
        prefix_v[pl.ds(16 * i, 16)] = (inc - v) + carry
        carry = carry + jnp.sum(v)

    def tokstep(i, c):
        off = i * 16
        flat = base + off + lax.iota(jnp.int32, 16)
        tok_v[pl.ds(off, 16)] = lax.shift_right_logical(flat, 3)
        return c

    lax.fori_loop(0, CHUNK // 16, tokstep, 0, unroll=4)
    cpk.wait()

    def step(i, c):
        off = i * 16
        pk = pk_v[pl.ds(off, 16)]
        e = pk & 63
        occ = lax.shift_right_logical(pk, 6)
        pre = plsc.load_gather(prefix_v, [e])
        rank_v[pl.ds(off, 16)] = pre + occ
        return c

    lax.fori_loop(0, CHUNK // 16, step, 0, unroll=4)
    csc.wait()
    s1 = pltpu.make_async_copy(sc_v, sh_sc.at[rank_v], sem_a)
    s1.start()
    s2 = pltpu.make_async_copy(tok_v, sh_tok.at[rank_v], sem_b)
    s2.start()
    s1.wait()
    s2.wait()
    plsc.subcore_barrier()
    ob = cid * (FLAT // 2) + sid * OUTW
    o1 = pltpu.make_async_copy(sh_sc.at[pl.ds(ob, OUTW)],
                               out_sc.at[pl.ds(ob, OUTW)], sem_a)
    o1.start()
    o2 = pltpu.make_async_copy(sh_tok.at[pl.ds(ob, OUTW)],
                               out_tok.at[pl.ds(ob, OUTW)], sem_b)
    o2.start()
    o1.wait()
    o2.wait()


@functools.cache
def _sc_scatter():
    return functools.partial(
        pl.kernel,
        mesh=plsc.VectorSubcoreMesh(core_axis_name="c", subcore_axis_name="s"),
        out_type=[
            jax.ShapeDtypeStruct((FLAT,), jnp.float32),
            jax.ShapeDtypeStruct((FLAT,), jnp.int32),
            jax.ShapeDtypeStruct((E,), jnp.int32),
        ],
        scratch_types=[
            pltpu.VMEM((CHUNK,), jnp.int32),
            pltpu.VMEM((CHUNK,), jnp.float32),
            pltpu.VMEM((CHUNK,), jnp.int32),
            pltpu.VMEM((CHUNK,), jnp.int32),
            pltpu.VMEM((E,), jnp.int32),
            pltpu.VMEM((E,), jnp.int32),
            pltpu.VMEM_SHARED((FLAT,), jnp.float32),
            pltpu.VMEM_SHARED((FLAT,), jnp.int32),
            pltpu.SemaphoreType.DMA,
            pltpu.SemaphoreType.DMA,
        ],
        compiler_params=pltpu.CompilerParams(needs_layout_passes=False),
    )(_sc_body)


def kernel(x, W_gate):
    topn, packed, counts2 = _router(x, W_gate)
    out_sc, out_tok, counts = _sc_scatter()(
        packed.reshape(FLAT), topn.reshape(FLAT), counts2.reshape(E))
    return out_sc, out_tok, counts


# speedup vs baseline: 1.2689x; 1.2593x over previous
import functools

import jax
import jax.numpy as jnp
from jax import lax
from jax.experimental import pallas as pl
from jax.experimental.pallas import tpu as pltpu
from jax.experimental.pallas import tpu_sc as plsc

DIM = 4096
E = 64
K = 8
N = 16384
BLK = 512
GRID = N // BLK
FLAT = N * K
NW = 32
SUB = 16
CHUNK = FLAT // SUB
OUTW = FLAT // NW


def _router_block(x_ref, w_ref, topn_ref, packed_ref, counts_ref,
                  carry_ref):
    b = pl.program_id(0)

    @pl.when(b == 0)
    def _():
        carry_ref[...] = jnp.zeros_like(carry_ref)

    logits = jax.lax.dot_general(
        w_ref[...], x_ref[...], (((1,), (1,)), ((), ())),
        preferred_element_type=jnp.float32)
    m = jnp.max(logits, axis=0, keepdims=True)
    p = jnp.exp(logits - m)
    probs = p / jnp.sum(p, axis=0, keepdims=True)

    row = lax.broadcasted_iota(jnp.int32, (E, BLK), 0)
    cur = probs
    vals, sels, ohs = [], [], []
    for _ in range(K):
        mx = jnp.max(cur, axis=0, keepdims=True)
        hit = cur == mx
        am = jnp.min(jnp.where(hit, row, E), axis=0, keepdims=True)
        oh = row == am
        vals.append(mx)
        sels.append(am)
        ohs.append(oh)
        cur = jnp.where(oh, -1.0, cur)

    top = jnp.concatenate(vals, axis=0)
    denom = jnp.clip(jnp.sum(top, axis=0, keepdims=True), 1e-8, None)
    topn_ref[...] = top / denom
    sel = jnp.concatenate(sels, axis=0)

    hot = jnp.zeros((E, BLK), jnp.float32)
    for oh in ohs:
        hot = hot + oh.astype(jnp.float32)
    ri = lax.broadcasted_iota(jnp.int32, (BLK, BLK), 0)
    ci = lax.broadcasted_iota(jnp.int32, (BLK, BLK), 1)
    utri = (ri < ci).astype(jnp.float32)
    within = jnp.dot(hot, utri, preferred_element_type=jnp.float32)
    ctot = within + carry_ref[...]
    occ_rows = [jnp.sum(jnp.where(oh, ctot, 0.0), axis=0, keepdims=True)
                for oh in ohs]
    occ = jnp.concatenate(occ_rows, axis=0).astype(jnp.int32)
    packed_ref[...] = (occ << 6) | sel
    carry_ref[...] = carry_ref[...] + jnp.sum(hot, axis=1, keepdims=True)
    counts_ref[...] = carry_ref[...].astype(jnp.int32)


def _router(x, wt):
    return pl.pallas_call(
        _router_block,
        grid=(GRID,),
        in_specs=[
            pl.BlockSpec((BLK, DIM), lambda b: (b, 0)),
            pl.BlockSpec((E, DIM), lambda b: (0, 0)),
        ],
        out_specs=[
            pl.BlockSpec((K, BLK), lambda b: (0, b)),
            pl.BlockSpec((K, BLK), lambda b: (0, b)),
            pl.BlockSpec((E, 1), lambda b: (0, 0)),
        ],
        out_shape=[
            jax.ShapeDtypeStruct((K, N), jnp.float32),
            jax.ShapeDtypeStruct((K, N), jnp.int32),
            jax.ShapeDtypeStruct((E, 1), jnp.int32),
        ],
        scratch_shapes=[pltpu.VMEM((E, 1), jnp.float32)],
    )(x, wt)


def _sc_body(pk_hbm, sc_hbm, counts_hbm, out_sc, out_tok, counts_out,
             pk_v, sc_v, rank_v, tok_v, counts_v, prefix_v,
             sh_sc, sh_tok, sem_a, sem_b):
    cid = lax.axis_index("c")
    sid = lax.axis_index("s")
    base = sid * CHUNK
    cpk = pltpu.make_async_copy(pk_hbm.at[pl.ds(base, CHUNK)], pk_v, sem_a)
    cpk.start()
    csc = pltpu.make_async_copy(sc_hbm.at[pl.ds(base, CHUNK)], sc_v, sem_b)
    csc.start()
    pltpu.sync_copy(counts_hbm, counts_v)

    @pl.when(jnp.logical_and(cid == 0, sid == 0))
    def _():
        pltpu.sync_copy(counts_v, counts_out)

    carry = jnp.int32(0)
    for i in range(E // 16):
        v = counts_v[pl.ds(16 * i, 16)]
        inc = plsc.cumsum(v)
        prefix_v[pl.ds(16 * i, 16)] = (inc - v) + carry
        carry = carry + jnp.sum(v)

    def tokstep(i, c):
        off = i * 16
        flat = base + off + lax.iota(jnp.int32, 16)
        tok_v[pl.ds(off, 16)] = flat & (N - 1)
        return c

    lax.fori_loop(0, CHUNK // 16, tokstep, 0, unroll=4)
    cpk.wait()

    def step(i, c):
        off = i * 16
        pk = pk_v[pl.ds(off, 16)]
        e = pk & 63
        occ = lax.shift_right_logical(pk, 6)
        pre = plsc.load_gather(prefix_v, [e])
        rank_v[pl.ds(off, 16)] = pre + occ
        return c

    lax.fori_loop(0, CHUNK // 16, step, 0, unroll=4)
    csc.wait()
    s1 = pltpu.make_async_copy(sc_v, sh_sc.at[rank_v], sem_a)
    s1.start()
    s2 = pltpu.make_async_copy(tok_v, sh_tok.at[rank_v], sem_b)
    s2.start()
    s1.wait()
    s2.wait()
    plsc.subcore_barrier()
    ob = cid * (FLAT // 2) + sid * OUTW
    o1 = pltpu.make_async_copy(sh_sc.at[pl.ds(ob, OUTW)],
                               out_sc.at[pl.ds(ob, OUTW)], sem_a)
    o1.start()
    o2 = pltpu.make_async_copy(sh_tok.at[pl.ds(ob, OUTW)],
                               out_tok.at[pl.ds(ob, OUTW)], sem_b)
    o2.start()
    o1.wait()
    o2.wait()


@functools.cache
def _sc_scatter():
    return functools.partial(
        pl.kernel,
        mesh=plsc.VectorSubcoreMesh(core_axis_name="c", subcore_axis_name="s"),
        out_type=[
            jax.ShapeDtypeStruct((FLAT,), jnp.float32),
            jax.ShapeDtypeStruct((FLAT,), jnp.int32),
            jax.ShapeDtypeStruct((E,), jnp.int32),
        ],
        scratch_types=[
            pltpu.VMEM((CHUNK,), jnp.int32),
            pltpu.VMEM((CHUNK,), jnp.float32),
            pltpu.VMEM((CHUNK,), jnp.int32),
            pltpu.VMEM((CHUNK,), jnp.int32),
            pltpu.VMEM((E,), jnp.int32),
            pltpu.VMEM((E,), jnp.int32),
            pltpu.VMEM_SHARED((FLAT,), jnp.float32),
            pltpu.VMEM_SHARED((FLAT,), jnp.int32),
            pltpu.SemaphoreType.DMA,
            pltpu.SemaphoreType.DMA,
        ],
        compiler_params=pltpu.CompilerParams(needs_layout_passes=False),
    )(_sc_body)


def kernel(x, W_gate):
    topn, packed, counts2 = _router(x, W_gate)
    out_sc, out_tok, counts = _sc_scatter()(
        packed.reshape(FLAT), topn.reshape(FLAT), counts2.reshape(E))
    return out_sc, out_tok, counts


# speedup vs baseline: 1.3741x; 1.0829x over previous
import functools

import jax
import jax.numpy as jnp
from jax import lax
from jax.experimental import pallas as pl
from jax.experimental.pallas import tpu as pltpu
from jax.experimental.pallas import tpu_sc as plsc

DIM = 4096
E = 64
K = 8
N = 16384
BLK = 1024
GRID = N // BLK
FLAT = N * K
NW = 32
SUB = 16
CHUNK = FLAT // SUB
OUTW = FLAT // NW


def _router_block(x_ref, w_ref, topn_ref, packed_ref, counts_ref,
                  carry_ref):
    b = pl.program_id(0)

    @pl.when(b == 0)
    def _():
        carry_ref[...] = jnp.zeros_like(carry_ref)

    logits = jax.lax.dot_general(
        w_ref[...], x_ref[...], (((1,), (1,)), ((), ())),
        preferred_element_type=jnp.float32)
    m = jnp.max(logits, axis=0, keepdims=True)
    p = jnp.exp(logits - m)
    probs = p / jnp.sum(p, axis=0, keepdims=True)

    row = lax.broadcasted_iota(jnp.int32, (E, BLK), 0)
    cur = probs
    vals, sels, ohs = [], [], []
    for _ in range(K):
        mx = jnp.max(cur, axis=0, keepdims=True)
        hit = cur == mx
        am = jnp.min(jnp.where(hit, row, E), axis=0, keepdims=True)
        oh = row == am
        vals.append(mx)
        sels.append(am)
        ohs.append(oh)
        cur = jnp.where(oh, -1.0, cur)

    top = jnp.concatenate(vals, axis=0)
    denom = jnp.clip(jnp.sum(top, axis=0, keepdims=True), 1e-8, None)
    topn_ref[...] = top / denom
    sel = jnp.concatenate(sels, axis=0)

    hot = jnp.zeros((E, BLK), jnp.float32)
    for oh in ohs:
        hot = hot + oh.astype(jnp.float32)
    ri = lax.broadcasted_iota(jnp.int32, (BLK, BLK), 0)
    ci = lax.broadcasted_iota(jnp.int32, (BLK, BLK), 1)
    utri = (ri < ci).astype(jnp.float32)
    within = jnp.dot(hot, utri, preferred_element_type=jnp.float32)
    ctot = within + carry_ref[...]
    occ_rows = [jnp.sum(jnp.where(oh, ctot, 0.0), axis=0, keepdims=True)
                for oh in ohs]
    occ = jnp.concatenate(occ_rows, axis=0).astype(jnp.int32)
    packed_ref[...] = (occ << 6) | sel
    carry_ref[...] = carry_ref[...] + jnp.sum(hot, axis=1, keepdims=True)
    counts_ref[...] = carry_ref[...].astype(jnp.int32)


def _router(x, wt):
    return pl.pallas_call(
        _router_block,
        grid=(GRID,),
        in_specs=[
            pl.BlockSpec((BLK, DIM), lambda b: (b, 0)),
            pl.BlockSpec((E, DIM), lambda b: (0, 0)),
        ],
        out_specs=[
            pl.BlockSpec((K, BLK), lambda b: (0, b)),
            pl.BlockSpec((K, BLK), lambda b: (0, b)),
            pl.BlockSpec((E, 1), lambda b: (0, 0)),
        ],
        out_shape=[
            jax.ShapeDtypeStruct((K, N), jnp.float32),
            jax.ShapeDtypeStruct((K, N), jnp.int32),
            jax.ShapeDtypeStruct((E, 1), jnp.int32),
        ],
        scratch_shapes=[pltpu.VMEM((E, 1), jnp.float32)],
    )(x, wt)


def _sc_body(pk_hbm, sc_hbm, counts_hbm, out_sc, out_tok, counts_out,
             pk_v, sc_v, rank_v, tok_v, counts_v, prefix_v,
             sh_sc, sh_tok, sem_a, sem_b):
    cid = lax.axis_index("c")
    sid = lax.axis_index("s")
    base = sid * CHUNK
    cpk = pltpu.make_async_copy(pk_hbm.at[pl.ds(base, CHUNK)], pk_v, sem_a)
    cpk.start()
    csc = pltpu.make_async_copy(sc_hbm.at[pl.ds(base, CHUNK)], sc_v, sem_b)
    csc.start()
    pltpu.sync_copy(counts_hbm, counts_v)

    @pl.when(jnp.logical_and(cid == 0, sid == 0))
    def _():
        pltpu.sync_copy(counts_v, counts_out)

    carry = jnp.int32(0)
    for i in range(E // 16):
        v = counts_v[pl.ds(16 * i, 16)]
        inc = plsc.cumsum(v)
        prefix_v[pl.ds(16 * i, 16)] = (inc - v) + carry
        carry = carry + jnp.sum(v)

    def tokstep(i, c):
        off = i * 16
        flat = base + off + lax.iota(jnp.int32, 16)
        tok_v[pl.ds(off, 16)] = flat & (N - 1)
        return c

    lax.fori_loop(0, CHUNK // 16, tokstep, 0, unroll=4)
    cpk.wait()

    def step(i, c):
        off = i * 16
        pk = pk_v[pl.ds(off, 16)]
        e = pk & 63
        occ = lax.shift_right_logical(pk, 6)
        pre = plsc.load_gather(prefix_v, [e])
        rank_v[pl.ds(off, 16)] = pre + occ
        return c

    lax.fori_loop(0, CHUNK // 16, step, 0, unroll=4)
    csc.wait()
    s1 = pltpu.make_async_copy(sc_v, sh_sc.at[rank_v], sem_a)
    s1.start()
    s2 = pltpu.make_async_copy(tok_v, sh_tok.at[rank_v], sem_b)
    s2.start()
    s1.wait()
    s2.wait()
    plsc.subcore_barrier()
    ob = cid * (FLAT // 2) + sid * OUTW
    o1 = pltpu.make_async_copy(sh_sc.at[pl.ds(ob, OUTW)],
                               out_sc.at[pl.ds(ob, OUTW)], sem_a)
    o1.start()
    o2 = pltpu.make_async_copy(sh_tok.at[pl.ds(ob, OUTW)],
                               out_tok.at[pl.ds(ob, OUTW)], sem_b)
    o2.start()
    o1.wait()
    o2.wait()


@functools.cache
def _sc_scatter():
    return functools.partial(
        pl.kernel,
        mesh=plsc.VectorSubcoreMesh(core_axis_name="c", subcore_axis_name="s"),
        out_type=[
            jax.ShapeDtypeStruct((FLAT,), jnp.float32),
            jax.ShapeDtypeStruct((FLAT,), jnp.int32),
            jax.ShapeDtypeStruct((E,), jnp.int32),
        ],
        scratch_types=[
            pltpu.VMEM((CHUNK,), jnp.int32),
            pltpu.VMEM((CHUNK,), jnp.float32),
            pltpu.VMEM((CHUNK,), jnp.int32),
            pltpu.VMEM((CHUNK,), jnp.int32),
            pltpu.VMEM((E,), jnp.int32),
            pltpu.VMEM((E,), jnp.int32),
            pltpu.VMEM_SHARED((FLAT,), jnp.float32),
            pltpu.VMEM_SHARED((FLAT,), jnp.int32),
            pltpu.SemaphoreType.DMA,
            pltpu.SemaphoreType.DMA,
        ],
        compiler_params=pltpu.CompilerParams(needs_layout_passes=False),
    )(_sc_body)


def kernel(x, W_gate):
    topn, packed, counts2 = _router(x, W_gate)
    out_sc, out_tok, counts = _sc_scatter()(
        packed.reshape(FLAT), topn.reshape(FLAT), counts2.reshape(E))
    return out_sc, out_tok, counts


# speedup vs baseline: 1.3774x; 1.0024x over previous
import functools

import jax
import jax.numpy as jnp
from jax import lax
from jax.experimental import pallas as pl
from jax.experimental.pallas import tpu as pltpu
from jax.experimental.pallas import tpu_sc as plsc

DIM = 4096
E = 64
K = 8
N = 16384
BLK = 1024
GRID = N // BLK
FLAT = N * K
NW = 32
SUB = 16
CHUNK = FLAT // SUB
OUTW = FLAT // NW


def _router_block(x_ref, w_ref, topn_ref, packed_ref, counts_ref,
                  carry_ref):
    b = pl.program_id(0)

    @pl.when(b == 0)
    def _():
        carry_ref[...] = jnp.zeros_like(carry_ref)

    logits = jax.lax.dot_general(
        w_ref[...], x_ref[...], (((1,), (1,)), ((), ())),
        preferred_element_type=jnp.float32)
    m = jnp.max(logits, axis=0, keepdims=True)
    p = jnp.exp(logits - m)
    probs = p / jnp.sum(p, axis=0, keepdims=True)

    row = lax.broadcasted_iota(jnp.int32, (E, BLK), 0)
    cur = probs
    vals, sels, ohs = [], [], []
    for _ in range(K):
        mx = jnp.max(cur, axis=0, keepdims=True)
        hit = cur == mx
        am = jnp.min(jnp.where(hit, row, E), axis=0, keepdims=True)
        oh = row == am
        vals.append(mx)
        sels.append(am)
        ohs.append(oh)
        cur = jnp.where(oh, -1.0, cur)

    top = jnp.concatenate(vals, axis=0)
    denom = jnp.clip(jnp.sum(top, axis=0, keepdims=True), 1e-8, None)
    topn_ref[...] = top / denom
    sel = jnp.concatenate(sels, axis=0)

    hot = jnp.zeros((E, BLK), jnp.float32)
    for oh in ohs:
        hot = hot + oh.astype(jnp.float32)
    ri = lax.broadcasted_iota(jnp.int32, (BLK, BLK), 0)
    ci = lax.broadcasted_iota(jnp.int32, (BLK, BLK), 1)
    utri = (ri < ci).astype(jnp.float32)
    within = jnp.dot(hot, utri, preferred_element_type=jnp.float32)
    ctot = within + carry_ref[...]
    occ_rows = [jnp.sum(jnp.where(oh, ctot, 0.0), axis=0, keepdims=True)
                for oh in ohs]
    occ = jnp.concatenate(occ_rows, axis=0).astype(jnp.int32)
    packed_ref[...] = (occ << 6) | sel
    carry_ref[...] = carry_ref[...] + jnp.sum(hot, axis=1, keepdims=True)
    counts_ref[...] = carry_ref[...].astype(jnp.int32)


def _router(x, wt):
    return pl.pallas_call(
        _router_block,
        grid=(GRID,),
        in_specs=[
            pl.BlockSpec((BLK, DIM), lambda b: (b, 0)),
            pl.BlockSpec((E, DIM), lambda b: (0, 0)),
        ],
        out_specs=[
            pl.BlockSpec((K, BLK), lambda b: (0, b)),
            pl.BlockSpec((K, BLK), lambda b: (0, b)),
            pl.BlockSpec((E, 1), lambda b: (0, 0)),
        ],
        out_shape=[
            jax.ShapeDtypeStruct((K, N), jnp.float32),
            jax.ShapeDtypeStruct((K, N), jnp.int32),
            jax.ShapeDtypeStruct((E, 1), jnp.int32),
        ],
        scratch_shapes=[pltpu.VMEM((E, 1), jnp.float32)],
    )(x, wt)


def _sc_body(pk_hbm, sc_hbm, counts_hbm, out_sc, out_tok, counts_out,
             pk_v, sc_v, rank_v, tok_v, cnt1_v, prefix_v,
             sh_sc, sh_tok, sem_a, sem_b):
    cid = lax.axis_index("c")
    sid = lax.axis_index("s")
    base = sid * CHUNK
    cpk = pltpu.make_async_copy(pk_hbm.at[pl.ds(base, CHUNK)], pk_v, sem_a)
    cpk.start()
    csc = pltpu.make_async_copy(sc_hbm.at[pl.ds(base, CHUNK)], sc_v, sem_b)
    csc.start()
    pltpu.sync_copy(counts_hbm, cnt1_v)

    carry = jnp.int32(0)
    for i in range(E // 16):
        v = cnt1_v[pl.ds(16 * i, 16)]
        inc = plsc.cumsum(v)
        prefix_v[pl.ds(16 * i, 16)] = (inc - v) + carry
        carry = carry + jnp.sum(v)

    @pl.when(jnp.logical_and(cid == 0, sid == 0))
    def _():
        pltpu.sync_copy(cnt1_v, counts_out)

    def tokstep(i, c):
        off = i * 16
        flat = base + off + lax.iota(jnp.int32, 16)
        tok_v[pl.ds(off, 16)] = flat & (N - 1)
        return c

    lax.fori_loop(0, CHUNK // 16, tokstep, 0, unroll=4)
    cpk.wait()

    def step(i, c):
        off = i * 16
        pk = pk_v[pl.ds(off, 16)]
        e = pk & 63
        occ = lax.shift_right_logical(pk, 6)
        pre = plsc.load_gather(prefix_v, [e])
        rank_v[pl.ds(off, 16)] = pre + occ
        return c

    lax.fori_loop(0, CHUNK // 16, step, 0, unroll=4)
    csc.wait()
    s1 = pltpu.make_async_copy(sc_v, sh_sc.at[rank_v], sem_a)
    s1.start()
    s2 = pltpu.make_async_copy(tok_v, sh_tok.at[rank_v], sem_b)
    s2.start()
    s1.wait()
    s2.wait()
    plsc.subcore_barrier()
    ob = cid * (FLAT // 2) + sid * OUTW
    o1 = pltpu.make_async_copy(sh_sc.at[pl.ds(ob, OUTW)],
                               out_sc.at[pl.ds(ob, OUTW)], sem_a)
    o1.start()
    o2 = pltpu.make_async_copy(sh_tok.at[pl.ds(ob, OUTW)],
                               out_tok.at[pl.ds(ob, OUTW)], sem_b)
    o2.start()
    o1.wait()
    o2.wait()


@functools.cache
def _sc_scatter():
    return functools.partial(
        pl.kernel,
        mesh=plsc.VectorSubcoreMesh(core_axis_name="c", subcore_axis_name="s"),
        out_type=[
            jax.ShapeDtypeStruct((FLAT,), jnp.float32),
            jax.ShapeDtypeStruct((FLAT,), jnp.int32),
            jax.ShapeDtypeStruct((E,), jnp.int32),
        ],
        scratch_types=[
            pltpu.VMEM((CHUNK,), jnp.int32),
            pltpu.VMEM((CHUNK,), jnp.float32),
            pltpu.VMEM((CHUNK,), jnp.int32),
            pltpu.VMEM((CHUNK,), jnp.int32),
            pltpu.VMEM((E,), jnp.int32),
            pltpu.VMEM((E,), jnp.int32),
            pltpu.VMEM_SHARED((FLAT,), jnp.float32),
            pltpu.VMEM_SHARED((FLAT,), jnp.int32),
            pltpu.SemaphoreType.DMA,
            pltpu.SemaphoreType.DMA,
        ],
        compiler_params=pltpu.CompilerParams(needs_layout_passes=False),
    )(_sc_body)


def kernel(x, W_gate):
    topn, packed, counts2 = _router(x, W_gate)
    out_sc, out_tok, counts = _sc_scatter()(
        packed.reshape(FLAT), topn.reshape(FLAT), counts2.reshape(E))
    return out_sc, out_tok, counts
